# baseline (speedup 1.0000x reference)
import functools

import jax
import jax.numpy as jnp
from jax import lax
from jax.experimental import pallas as pl
from jax.experimental.pallas import tpu as pltpu
from jax.experimental.pallas import tpu_sc as plsc

NC = 2
NS = 16
NW = NC * NS
L = 16
B_A = 64
B_B = 48
DEPTH = 4


def _sc_mesh():
    return plsc.VectorSubcoreMesh(
        core_axis_name="c", subcore_axis_name="s", num_cores=NC,
        num_subcores=NS)


def _make_pass(n_pad, ch, d, bsz, mode):
    rps = n_pad // NS
    dep = DEPTH
    assert ch % dep == 0

    scratch = []
    for _ in range(dep):
        scratch += [pltpu.VMEM((bsz,), jnp.int32),
                    pltpu.VMEM((bsz,), jnp.int32)]
    for _ in range(dep):
        scratch.append(pltpu.VMEM((bsz, d), jnp.float32))
    scratch.append(pltpu.VMEM((n_pad,), jnp.float32))
    if mode == 'b':
        scratch.append(pltpu.VMEM((n_pad,), jnp.float32))
    scratch.append(pltpu.VMEM_SHARED((n_pad, d), jnp.float32))
    for _ in range(dep):
        scratch.append(pltpu.SemaphoreType.DMA)

    def body(*refs):
        if mode == 'a':
            (x_hbm, src_hbm, dst_hbm, zero_d_hbm, zero_1_hbm,
             sums_out, vec_out) = refs[:7]
            scr = refs[7:]
        else:
            (x_hbm, inv_hbm, src_hbm, dst_hbm, zero_d_hbm, zero_1_hbm,
             sums_out, vec_out) = refs[:8]
            scr = refs[8:]
        idx_s = [scr[2 * t] for t in range(dep)]
        idx_d = [scr[2 * t + 1] for t in range(dep)]
        rows = list(scr[2 * dep:3 * dep])
        pos = 3 * dep
        vec_v = scr[pos]
        pos += 1
        if mode == 'b':
            inv_v = scr[pos]
            pos += 1
        sum_acc = scr[pos]
        pos += 1
        sems = list(scr[pos:pos + dep])

        c = lax.axis_index("c")
        s = lax.axis_index("s")
        wid = s * NC + c
        stripe = pl.ds(s * rps, rps)

        pltpu.sync_copy(zero_d_hbm.at[stripe], sum_acc.at[stripe])
        pltpu.sync_copy(zero_1_hbm, vec_v)
        if mode == 'b':
            pltpu.sync_copy(inv_hbm, inv_v)
        plsc.subcore_barrier()

        ones = jnp.ones((L,), jnp.float32)
        dummy = zero_d_hbm.at[pl.ds(0, bsz)]

        def regop(t):
            def grp(g, _):
                dv = idx_d[t][pl.ds(g * L, L)]
                if mode == 'a':
                    plsc.addupdate_scatter(vec_v, [dv], ones)
                else:
                    sv = idx_s[t][pl.ds(g * L, L)]
                    vals = plsc.load_gather(inv_v, [dv])
                    plsc.addupdate_scatter(vec_v, [sv], vals)
                return 0
            lax.fori_loop(0, bsz // L, grp, 0)

        def fetch(t, k):
            pltpu.sync_copy(src_hbm.at[wid, k], idx_s[t])
            pltpu.sync_copy(dst_hbm.at[wid, k], idx_d[t])
            pltpu.async_copy(x_hbm.at[idx_s[t]], rows[t], sems[t])

        for t in range(dep - 1):
            fetch(t, t)

        def block(q, _):
            for t in range(dep):
                k = q * dep + t
                fetch((t + dep - 1) % dep, lax.rem(k + dep - 1, ch))
                regop(t)
                pltpu.make_async_copy(dummy, rows[t], sems[t]).wait()
                pltpu.sync_copy(rows[t], sum_acc.at[idx_d[t]], add=True)
            return 0
        lax.fori_loop(0, ch // dep, block, 0)
        for t in range(dep - 1):
            pltpu.make_async_copy(dummy, rows[t], sems[t]).wait()

        plsc.subcore_barrier()
        pltpu.sync_copy(sum_acc.at[stripe], sums_out.at[c, stripe])
        pltpu.sync_copy(vec_v, vec_out.at[wid])

    return pl.kernel(
        body,
        out_type=[
            jax.ShapeDtypeStruct((NC, n_pad, d), jnp.float32),
            jax.ShapeDtypeStruct((NW, n_pad), jnp.float32),
        ],
        mesh=_sc_mesh(),
        compiler_params=pltpu.CompilerParams(needs_layout_passes=False),
        scratch_types=scratch,
    )


def _layer_body(make_inv, n, n_pad,
                x_ref, sums_ref, degs_ref, ws_ref, wn_ref, b_ref, g_ref,
                be_ref, *out_refs):
    x = x_ref[...]
    summed = sums_ref[0, :n, :] + sums_ref[1, :n, :]
    deg_full = jnp.sum(degs_ref[...], axis=0)
    deg = deg_full[:n, None]
    agg = jnp.where(deg > 0, summed / jnp.maximum(deg, 1.0), 0.0)
    t = (jnp.dot(x, ws_ref[...], preferred_element_type=jnp.float32)
         + jnp.dot(agg, wn_ref[...], preferred_element_type=jnp.float32)
         + b_ref[...])
    m = jnp.mean(t, axis=0, keepdims=True)
    v = jnp.mean(jnp.square(t - m), axis=0, keepdims=True)
    h = g_ref[...] * (t - m) * lax.rsqrt(v + 1e-5) + be_ref[...]
    out_refs[0][...] = jnp.maximum(h, 0.0)
    if make_inv:
        row = lax.iota(jnp.int32, n_pad)
        inv = jnp.where(row < n, 1.0 / jnp.maximum(deg_full, 1.0), 0.0)
        out_refs[1][...] = inv


def _final_body(n, feat_ref, cv_ref, ws_ref, wn_ref, b_ref, out_ref):
    feat = feat_ref[...]
    cvec = jnp.sum(cv_ref[...], axis=0)[:n, None]
    sacc = jnp.sum(feat * cvec, axis=0, keepdims=True)
    mf = jnp.mean(feat, axis=0, keepdims=True)
    o = (jnp.dot(mf, ws_ref[...], preferred_element_type=jnp.float32)
         + jnp.dot(sacc / n, wn_ref[...], preferred_element_type=jnp.float32)
         + b_ref[...])
    z = o - jnp.max(o, axis=-1, keepdims=True)
    out_ref[...] = z - jnp.log(jnp.sum(jnp.exp(z), axis=-1, keepdims=True))


def _pad_edges(edge_index, n, e, bsz):
    ch = -(-e // (NW * bsz))
    ch += (-ch) % DEPTH
    e_pad = ch * NW * bsz
    src = edge_index[0]
    dst = edge_index[1]
    pad = e_pad - e
    if pad:
        src = jnp.concatenate([src, jnp.zeros((pad,), jnp.int32)])
        dst = jnp.concatenate([dst, jnp.full((pad,), n, jnp.int32)])
    return src.reshape(NW, ch, bsz), dst.reshape(NW, ch, bsz), ch


def kernel(edge_index, inputs, W_self0, W_neigh0, b0, gamma0, beta0,
           W_self1, W_neigh1, b1, gamma1, beta1, W_self2, W_neigh2, b2):
    n, d = inputs.shape
    e = edge_index.shape[1]
    d_out = W_self2.shape[1]

    n_pad = -(-(n + 1) // (NS * 8)) * (NS * 8)
    src_a, dst_a, ch_a = _pad_edges(edge_index, n, e, B_A)
    src_b, dst_b, ch_b = _pad_edges(edge_index, n, e, B_B)
    zero_d = jnp.zeros((n_pad, d), jnp.float32)
    zero_1 = jnp.zeros((n_pad,), jnp.float32)

    pass_a = _make_pass(n_pad, ch_a, d, B_A, 'a')
    pass_b = _make_pass(n_pad, ch_b, d, B_B, 'b')

    def layer(x, sums, degs, ws, wn, b, g, be, make_inv):
        outs = [jax.ShapeDtypeStruct((n, d), jnp.float32)]
        if make_inv:
            outs.append(jax.ShapeDtypeStruct((n_pad,), jnp.float32))
        return pl.pallas_call(
            functools.partial(_layer_body, make_inv, n, n_pad),
            out_shape=outs,
        )(x, sums, degs, ws, wn, b, g, be)

    sums_a, degv = pass_a(inputs, src_a, dst_a, zero_d, zero_1)
    h1, invd = layer(inputs, sums_a, degv, W_self0, W_neigh0, b0, gamma0,
                     beta0, True)
    sums_b, cv = pass_b(h1, invd, src_b, dst_b, zero_d, zero_1)
    (feat,) = layer(h1, sums_b, degv, W_self1, W_neigh1, b1, gamma1,
                    beta1, False)
    out = pl.pallas_call(
        functools.partial(_final_body, n),
        out_shape=jax.ShapeDtypeStruct((1, d_out), jnp.float32),
    )(feat, cv, W_self2, W_neigh2, b2)
    return out, inputs, feat


# speedup vs baseline: 1.1012x; 1.1012x over previous
import functools

import jax
import jax.numpy as jnp
from jax import lax
from jax.experimental import pallas as pl
from jax.experimental.pallas import tpu as pltpu
from jax.experimental.pallas import tpu_sc as plsc

NC = 2
NS = 16
NW = NC * NS
L = 16
B_A = 64
B_B = 64
DEPTH = 3


def _sc_mesh():
    return plsc.VectorSubcoreMesh(
        core_axis_name="c", subcore_axis_name="s", num_cores=NC,
        num_subcores=NS)


def _make_pass(n_pad, ch, d, bsz, mode):
    rps = n_pad // NS
    dep = DEPTH
    assert ch % dep == 0

    scratch = []
    for _ in range(dep):
        scratch += [pltpu.VMEM((bsz,), jnp.int32),
                    pltpu.VMEM((bsz,), jnp.int32)]
    for _ in range(dep):
        scratch.append(pltpu.VMEM((bsz, d), jnp.float32))
    scratch.append(pltpu.VMEM((n_pad,), jnp.float32))
    if mode == 'b':
        scratch.append(pltpu.VMEM((n_pad,), jnp.float32))
    scratch.append(pltpu.VMEM_SHARED((n_pad, d), jnp.float32))
    for _ in range(dep):
        scratch.append(pltpu.SemaphoreType.DMA)

    def body(*refs):
        if mode == 'a':
            (x_hbm, src_hbm, dst_hbm, zero_d_hbm, zero_1_hbm,
             sums_out, vec_out) = refs[:7]
            scr = refs[7:]
        else:
            (x_hbm, inv_hbm, src_hbm, dst_hbm, zero_d_hbm, zero_1_hbm,
             sums_out, vec_out) = refs[:8]
            scr = refs[8:]
        idx_s = [scr[2 * t] for t in range(dep)]
        idx_d = [scr[2 * t + 1] for t in range(dep)]
        rows = list(scr[2 * dep:3 * dep])
        pos = 3 * dep
        vec_v = scr[pos]
        pos += 1
        if mode == 'b':
            inv_v = scr[pos]
            pos += 1
        sum_acc = scr[pos]
        pos += 1
        sems = list(scr[pos:pos + dep])

        c = lax.axis_index("c")
        s = lax.axis_index("s")
        wid = s * NC + c
        stripe = pl.ds(s * rps, rps)

        pltpu.sync_copy(zero_d_hbm.at[stripe], sum_acc.at[stripe])
        pltpu.sync_copy(zero_1_hbm, vec_v)
        if mode == 'b':
            pltpu.sync_copy(inv_hbm, inv_v)
        plsc.subcore_barrier()

        ones = jnp.ones((L,), jnp.float32)
        dummy = zero_d_hbm.at[pl.ds(0, bsz)]

        def regop(t):
            def grp(g, _):
                dv = idx_d[t][pl.ds(g * L, L)]
                if mode == 'a':
                    plsc.addupdate_scatter(vec_v, [dv], ones)
                else:
                    sv = idx_s[t][pl.ds(g * L, L)]
                    vals = plsc.load_gather(inv_v, [dv])
                    plsc.addupdate_scatter(vec_v, [sv], vals)
                return 0
            lax.fori_loop(0, bsz // L, grp, 0)

        def fetch(t, k):
            pltpu.sync_copy(src_hbm.at[wid, k], idx_s[t])
            pltpu.sync_copy(dst_hbm.at[wid, k], idx_d[t])
            pltpu.async_copy(x_hbm.at[idx_s[t]], rows[t], sems[t])

        for t in range(dep - 1):
            fetch(t, t)

        def block(q, _):
            for t in range(dep):
                k = q * dep + t
                fetch((t + dep - 1) % dep, lax.rem(k + dep - 1, ch))
                regop(t)
                pltpu.make_async_copy(dummy, rows[t], sems[t]).wait()
                pltpu.sync_copy(rows[t], sum_acc.at[idx_d[t]], add=True)
            return 0
        lax.fori_loop(0, ch // dep, block, 0)
        for t in range(dep - 1):
            pltpu.make_async_copy(dummy, rows[t], sems[t]).wait()

        plsc.subcore_barrier()
        pltpu.sync_copy(sum_acc.at[stripe], sums_out.at[c, stripe])
        pltpu.sync_copy(vec_v, vec_out.at[wid])

    return pl.kernel(
        body,
        out_type=[
            jax.ShapeDtypeStruct((NC, n_pad, d), jnp.float32),
            jax.ShapeDtypeStruct((NW, n_pad), jnp.float32),
        ],
        mesh=_sc_mesh(),
        compiler_params=pltpu.CompilerParams(needs_layout_passes=False),
        scratch_types=scratch,
    )


def _layer_body(make_inv, n, n_pad,
                x_ref, sums_ref, degs_ref, ws_ref, wn_ref, b_ref, g_ref,
                be_ref, *out_refs):
    x = x_ref[...]
    summed = sums_ref[0, :n, :] + sums_ref[1, :n, :]
    deg_full = jnp.sum(degs_ref[...], axis=0)
    deg = deg_full[:n, None]
    agg = jnp.where(deg > 0, summed / jnp.maximum(deg, 1.0), 0.0)
    t = (jnp.dot(x, ws_ref[...], preferred_element_type=jnp.float32)
         + jnp.dot(agg, wn_ref[...], preferred_element_type=jnp.float32)
         + b_ref[...])
    m = jnp.mean(t, axis=0, keepdims=True)
    v = jnp.mean(jnp.square(t - m), axis=0, keepdims=True)
    h = g_ref[...] * (t - m) * lax.rsqrt(v + 1e-5) + be_ref[...]
    out_refs[0][...] = jnp.maximum(h, 0.0)
    if make_inv:
        row = lax.iota(jnp.int32, n_pad)
        inv = jnp.where(row < n, 1.0 / jnp.maximum(deg_full, 1.0), 0.0)
        out_refs[1][...] = inv


def _final_body(n, feat_ref, cv_ref, ws_ref, wn_ref, b_ref, out_ref):
    feat = feat_ref[...]
    cvec = jnp.sum(cv_ref[...], axis=0)[:n, None]
    sacc = jnp.sum(feat * cvec, axis=0, keepdims=True)
    mf = jnp.mean(feat, axis=0, keepdims=True)
    o = (jnp.dot(mf, ws_ref[...], preferred_element_type=jnp.float32)
         + jnp.dot(sacc / n, wn_ref[...], preferred_element_type=jnp.float32)
         + b_ref[...])
    z = o - jnp.max(o, axis=-1, keepdims=True)
    out_ref[...] = z - jnp.log(jnp.sum(jnp.exp(z), axis=-1, keepdims=True))


def _pad_edges(edge_index, n, e, bsz):
    ch = -(-e // (NW * bsz))
    ch += (-ch) % DEPTH
    e_pad = ch * NW * bsz
    src = edge_index[0]
    dst = edge_index[1]
    pad = e_pad - e
    if pad:
        src = jnp.concatenate([src, jnp.zeros((pad,), jnp.int32)])
        dst = jnp.concatenate([dst, jnp.full((pad,), n, jnp.int32)])
    return src.reshape(NW, ch, bsz), dst.reshape(NW, ch, bsz), ch


def kernel(edge_index, inputs, W_self0, W_neigh0, b0, gamma0, beta0,
           W_self1, W_neigh1, b1, gamma1, beta1, W_self2, W_neigh2, b2):
    n, d = inputs.shape
    e = edge_index.shape[1]
    d_out = W_self2.shape[1]

    n_pad = -(-(n + 1) // (NS * 8)) * (NS * 8)
    src_a, dst_a, ch_a = _pad_edges(edge_index, n, e, B_A)
    src_b, dst_b, ch_b = _pad_edges(edge_index, n, e, B_B)
    zero_d = jnp.zeros((n_pad, d), jnp.float32)
    zero_1 = jnp.zeros((n_pad,), jnp.float32)

    pass_a = _make_pass(n_pad, ch_a, d, B_A, 'a')
    pass_b = _make_pass(n_pad, ch_b, d, B_B, 'b')

    def layer(x, sums, degs, ws, wn, b, g, be, make_inv):
        outs = [jax.ShapeDtypeStruct((n, d), jnp.float32)]
        if make_inv:
            outs.append(jax.ShapeDtypeStruct((n_pad,), jnp.float32))
        return pl.pallas_call(
            functools.partial(_layer_body, make_inv, n, n_pad),
            out_shape=outs,
        )(x, sums, degs, ws, wn, b, g, be)

    sums_a, degv = pass_a(inputs, src_a, dst_a, zero_d, zero_1)
    h1, invd = layer(inputs, sums_a, degv, W_self0, W_neigh0, b0, gamma0,
                     beta0, True)
    sums_b, cv = pass_b(h1, invd, src_b, dst_b, zero_d, zero_1)
    (feat,) = layer(h1, sums_b, degv, W_self1, W_neigh1, b1, gamma1,
                    beta1, False)
    out = pl.pallas_call(
        functools.partial(_final_body, n),
        out_shape=jax.ShapeDtypeStruct((1, d_out), jnp.float32),
    )(feat, cv, W_self2, W_neigh2, b2)
    return out, inputs, feat


# speedup vs baseline: 1.1184x; 1.0156x over previous
import functools

import jax
import jax.numpy as jnp
from jax import lax
from jax.experimental import pallas as pl
from jax.experimental.pallas import tpu as pltpu
from jax.experimental.pallas import tpu_sc as plsc

NC = 2
NS = 16
NW = NC * NS
L = 16
B_A = 128
B_B = 64
DEPTH = 2


def _sc_mesh():
    return plsc.VectorSubcoreMesh(
        core_axis_name="c", subcore_axis_name="s", num_cores=NC,
        num_subcores=NS)


def _make_pass(n_pad, ch, d, bsz, mode):
    rps = n_pad // NS
    dep = DEPTH
    assert ch % dep == 0

    scratch = []
    for _ in range(dep):
        scratch += [pltpu.VMEM((bsz,), jnp.int32),
                    pltpu.VMEM((bsz,), jnp.int32)]
    for _ in range(dep):
        scratch.append(pltpu.VMEM((bsz, d), jnp.float32))
    scratch.append(pltpu.VMEM((n_pad,), jnp.float32))
    if mode == 'b':
        scratch.append(pltpu.VMEM((n_pad,), jnp.float32))
    scratch.append(pltpu.VMEM_SHARED((n_pad, d), jnp.float32))
    for _ in range(dep):
        scratch.append(pltpu.SemaphoreType.DMA)

    def body(*refs):
        if mode == 'a':
            (x_hbm, src_hbm, dst_hbm, zero_d_hbm, zero_1_hbm,
             sums_out, vec_out) = refs[:7]
            scr = refs[7:]
        else:
            (x_hbm, inv_hbm, src_hbm, dst_hbm, zero_d_hbm, zero_1_hbm,
             sums_out, vec_out) = refs[:8]
            scr = refs[8:]
        idx_s = [scr[2 * t] for t in range(dep)]
        idx_d = [scr[2 * t + 1] for t in range(dep)]
        rows = list(scr[2 * dep:3 * dep])
        pos = 3 * dep
        vec_v = scr[pos]
        pos += 1
        if mode == 'b':
            inv_v = scr[pos]
            pos += 1
        sum_acc = scr[pos]
        pos += 1
        sems = list(scr[pos:pos + dep])

        c = lax.axis_index("c")
        s = lax.axis_index("s")
        wid = s * NC + c
        stripe = pl.ds(s * rps, rps)

        pltpu.sync_copy(zero_d_hbm.at[stripe], sum_acc.at[stripe])
        pltpu.sync_copy(zero_1_hbm, vec_v)
        if mode == 'b':
            pltpu.sync_copy(inv_hbm, inv_v)
        plsc.subcore_barrier()

        ones = jnp.ones((L,), jnp.float32)
        dummy = zero_d_hbm.at[pl.ds(0, bsz)]

        def regop(t):
            def grp(g, _):
                dv = idx_d[t][pl.ds(g * L, L)]
                if mode == 'a':
                    plsc.addupdate_scatter(vec_v, [dv], ones)
                else:
                    sv = idx_s[t][pl.ds(g * L, L)]
                    vals = plsc.load_gather(inv_v, [dv])
                    plsc.addupdate_scatter(vec_v, [sv], vals)
                return 0
            lax.fori_loop(0, bsz // L, grp, 0)

        def fetch(t, k):
            pltpu.sync_copy(src_hbm.at[wid, k], idx_s[t])
            pltpu.sync_copy(dst_hbm.at[wid, k], idx_d[t])
            pltpu.async_copy(x_hbm.at[idx_s[t]], rows[t], sems[t])

        for t in range(dep - 1):
            fetch(t, t)

        def block(q, _):
            for t in range(dep):
                k = q * dep + t
                fetch((t + dep - 1) % dep, lax.rem(k + dep - 1, ch))
                regop(t)
                pltpu.make_async_copy(dummy, rows[t], sems[t]).wait()
                pltpu.sync_copy(rows[t], sum_acc.at[idx_d[t]], add=True)
            return 0
        lax.fori_loop(0, ch // dep, block, 0)
        for t in range(dep - 1):
            pltpu.make_async_copy(dummy, rows[t], sems[t]).wait()

        plsc.subcore_barrier()
        pltpu.sync_copy(sum_acc.at[stripe], sums_out.at[c, stripe])
        pltpu.sync_copy(vec_v, vec_out.at[wid])

    return pl.kernel(
        body,
        out_type=[
            jax.ShapeDtypeStruct((NC, n_pad, d), jnp.float32),
            jax.ShapeDtypeStruct((NW, n_pad), jnp.float32),
        ],
        mesh=_sc_mesh(),
        compiler_params=pltpu.CompilerParams(needs_layout_passes=False),
        scratch_types=scratch,
    )


def _layer_body(make_inv, n, n_pad,
                x_ref, sums_ref, degs_ref, ws_ref, wn_ref, b_ref, g_ref,
                be_ref, *out_refs):
    x = x_ref[...]
    summed = sums_ref[0, :n, :] + sums_ref[1, :n, :]
    deg_full = jnp.sum(degs_ref[...], axis=0)
    deg = deg_full[:n, None]
    agg = jnp.where(deg > 0, summed / jnp.maximum(deg, 1.0), 0.0)
    t = (jnp.dot(x, ws_ref[...], preferred_element_type=jnp.float32)
         + jnp.dot(agg, wn_ref[...], preferred_element_type=jnp.float32)
         + b_ref[...])
    m = jnp.mean(t, axis=0, keepdims=True)
    v = jnp.mean(jnp.square(t - m), axis=0, keepdims=True)
    h = g_ref[...] * (t - m) * lax.rsqrt(v + 1e-5) + be_ref[...]
    out_refs[0][...] = jnp.maximum(h, 0.0)
    if make_inv:
        row = lax.iota(jnp.int32, n_pad)
        inv = jnp.where(row < n, 1.0 / jnp.maximum(deg_full, 1.0), 0.0)
        out_refs[1][...] = inv


def _final_body(n, feat_ref, cv_ref, ws_ref, wn_ref, b_ref, out_ref):
    feat = feat_ref[...]
    cvec = jnp.sum(cv_ref[...], axis=0)[:n, None]
    sacc = jnp.sum(feat * cvec, axis=0, keepdims=True)
    mf = jnp.mean(feat, axis=0, keepdims=True)
    o = (jnp.dot(mf, ws_ref[...], preferred_element_type=jnp.float32)
         + jnp.dot(sacc / n, wn_ref[...], preferred_element_type=jnp.float32)
         + b_ref[...])
    z = o - jnp.max(o, axis=-1, keepdims=True)
    out_ref[...] = z - jnp.log(jnp.sum(jnp.exp(z), axis=-1, keepdims=True))


def _pad_edges(edge_index, n, e, bsz):
    ch = -(-e // (NW * bsz))
    ch += (-ch) % DEPTH
    e_pad = ch * NW * bsz
    src = edge_index[0]
    dst = edge_index[1]
    pad = e_pad - e
    if pad:
        src = jnp.concatenate([src, jnp.zeros((pad,), jnp.int32)])
        dst = jnp.concatenate([dst, jnp.full((pad,), n, jnp.int32)])
    return src.reshape(NW, ch, bsz), dst.reshape(NW, ch, bsz), ch


def kernel(edge_index, inputs, W_self0, W_neigh0, b0, gamma0, beta0,
           W_self1, W_neigh1, b1, gamma1, beta1, W_self2, W_neigh2, b2):
    n, d = inputs.shape
    e = edge_index.shape[1]
    d_out = W_self2.shape[1]

    n_pad = -(-(n + 1) // (NS * 8)) * (NS * 8)
    src_a, dst_a, ch_a = _pad_edges(edge_index, n, e, B_A)
    src_b, dst_b, ch_b = _pad_edges(edge_index, n, e, B_B)
    zero_d = jnp.zeros((n_pad, d), jnp.float32)
    zero_1 = jnp.zeros((n_pad,), jnp.float32)

    pass_a = _make_pass(n_pad, ch_a, d, B_A, 'a')
    pass_b = _make_pass(n_pad, ch_b, d, B_B, 'b')

    def layer(x, sums, degs, ws, wn, b, g, be, make_inv):
        outs = [jax.ShapeDtypeStruct((n, d), jnp.float32)]
        if make_inv:
            outs.append(jax.ShapeDtypeStruct((n_pad,), jnp.float32))
        return pl.pallas_call(
            functools.partial(_layer_body, make_inv, n, n_pad),
            out_shape=outs,
        )(x, sums, degs, ws, wn, b, g, be)

    sums_a, degv = pass_a(inputs, src_a, dst_a, zero_d, zero_1)
    h1, invd = layer(inputs, sums_a, degv, W_self0, W_neigh0, b0, gamma0,
                     beta0, True)
    sums_b, cv = pass_b(h1, invd, src_b, dst_b, zero_d, zero_1)
    (feat,) = layer(h1, sums_b, degv, W_self1, W_neigh1, b1, gamma1,
                    beta1, False)
    out = pl.pallas_call(
        functools.partial(_final_body, n),
        out_shape=jax.ShapeDtypeStruct((1, d_out), jnp.float32),
    )(feat, cv, W_self2, W_neigh2, b2)
    return out, inputs, feat


# speedup vs baseline: 1.3772x; 1.2314x over previous
import functools

import jax
import jax.numpy as jnp
from jax import lax
from jax.experimental import pallas as pl
from jax.experimental.pallas import tpu as pltpu
from jax.experimental.pallas import tpu_sc as plsc

NC = 2
NS = 16
NW = NC * NS
L = 16
B_A = 64
B_B = 64
DEPTH = 2


def _sc_mesh():
    return plsc.VectorSubcoreMesh(
        core_axis_name="c", subcore_axis_name="s", num_cores=NC,
        num_subcores=NS)


def _make_pass(n_pad, ch, d, bsz, mode):
    rps = n_pad // NS
    dep = DEPTH
    assert ch % dep == 0

    scratch = []
    for _ in range(dep):
        scratch.append(pltpu.VMEM((2, bsz), jnp.int32))
    for _ in range(dep):
        scratch.append(pltpu.VMEM((bsz, d), jnp.float32))
    scratch.append(pltpu.VMEM((n_pad,), jnp.float32))
    if mode == 'b':
        scratch.append(pltpu.VMEM((n_pad,), jnp.float32))
    scratch.append(pltpu.VMEM_SHARED((n_pad, d), jnp.float32))
    for _ in range(dep):
        scratch.append(pltpu.SemaphoreType.DMA)

    def body(*refs):
        if mode == 'a':
            (x_hbm, edges_hbm, zero_d_hbm, zero_1_hbm,
             sums_out, vec_out) = refs[:6]
            scr = refs[6:]
        else:
            (x_hbm, inv_hbm, edges_hbm, zero_d_hbm, zero_1_hbm,
             sums_out, vec_out) = refs[:7]
            scr = refs[7:]
        idxb = list(scr[:dep])
        rows = list(scr[dep:2 * dep])
        pos = 2 * dep
        vec_v = scr[pos]
        pos += 1
        if mode == 'b':
            inv_v = scr[pos]
            pos += 1
        sum_acc = scr[pos]
        pos += 1
        sems = list(scr[pos:pos + dep])

        c = lax.axis_index("c")
        s = lax.axis_index("s")
        wid = s * NC + c
        stripe = pl.ds(s * rps, rps)

        pltpu.sync_copy(zero_d_hbm.at[stripe], sum_acc.at[stripe])
        pltpu.sync_copy(zero_1_hbm, vec_v)
        if mode == 'b':
            pltpu.sync_copy(inv_hbm, inv_v)
        plsc.subcore_barrier()

        ones = jnp.ones((L,), jnp.float32)
        dummy = zero_d_hbm.at[pl.ds(0, bsz)]

        def regop(t):
            def grp(g, _):
                dv = idxb[t][1, pl.ds(g * L, L)]
                if mode == 'a':
                    plsc.addupdate_scatter(vec_v, [dv], ones)
                else:
                    sv = idxb[t][0, pl.ds(g * L, L)]
                    vals = plsc.load_gather(inv_v, [dv])
                    plsc.addupdate_scatter(vec_v, [sv], vals)
                return 0
            lax.fori_loop(0, bsz // L, grp, 0)

        def fetch(t, k):
            pltpu.sync_copy(edges_hbm.at[wid, k], idxb[t])
            pltpu.async_copy(x_hbm.at[idxb[t].at[0]], rows[t], sems[t])

        for t in range(dep - 1):
            fetch(t, t)

        def block(q, _):
            for t in range(dep):
                k = q * dep + t
                fetch((t + dep - 1) % dep, lax.rem(k + dep - 1, ch))
                regop(t)
                pltpu.make_async_copy(dummy, rows[t], sems[t]).wait()
                pltpu.sync_copy(rows[t], sum_acc.at[idxb[t].at[1]], add=True)
            return 0
        lax.fori_loop(0, ch // dep, block, 0)
        for t in range(dep - 1):
            pltpu.make_async_copy(dummy, rows[t], sems[t]).wait()

        plsc.subcore_barrier()
        pltpu.sync_copy(sum_acc.at[stripe], sums_out.at[c, stripe])
        pltpu.sync_copy(vec_v, vec_out.at[wid])

    return pl.kernel(
        body,
        out_type=[
            jax.ShapeDtypeStruct((NC, n_pad, d), jnp.float32),
            jax.ShapeDtypeStruct((NW, n_pad), jnp.float32),
        ],
        mesh=_sc_mesh(),
        compiler_params=pltpu.CompilerParams(needs_layout_passes=False),
        scratch_types=scratch,
    )


def _layer_body(make_inv, n, n_pad,
                x_ref, sums_ref, degs_ref, ws_ref, wn_ref, b_ref, g_ref,
                be_ref, *out_refs):
    x = x_ref[...]
    summed = sums_ref[0, :n, :] + sums_ref[1, :n, :]
    deg_full = jnp.sum(degs_ref[...], axis=0)
    deg = deg_full[:n, None]
    agg = jnp.where(deg > 0, summed / jnp.maximum(deg, 1.0), 0.0)
    t = (jnp.dot(x, ws_ref[...], preferred_element_type=jnp.float32)
         + jnp.dot(agg, wn_ref[...], preferred_element_type=jnp.float32)
         + b_ref[...])
    m = jnp.mean(t, axis=0, keepdims=True)
    v = jnp.mean(jnp.square(t - m), axis=0, keepdims=True)
    h = g_ref[...] * (t - m) * lax.rsqrt(v + 1e-5) + be_ref[...]
    out_refs[0][...] = jnp.maximum(h, 0.0)
    if make_inv:
        row = lax.iota(jnp.int32, n_pad)
        inv = jnp.where(row < n, 1.0 / jnp.maximum(deg_full, 1.0), 0.0)
        out_refs[1][...] = inv


def _final_body(n, feat_ref, cv_ref, ws_ref, wn_ref, b_ref, out_ref):
    feat = feat_ref[...]
    cvec = jnp.sum(cv_ref[...], axis=0)[:n, None]
    sacc = jnp.sum(feat * cvec, axis=0, keepdims=True)
    mf = jnp.mean(feat, axis=0, keepdims=True)
    o = (jnp.dot(mf, ws_ref[...], preferred_element_type=jnp.float32)
         + jnp.dot(sacc / n, wn_ref[...], preferred_element_type=jnp.float32)
         + b_ref[...])
    z = o - jnp.max(o, axis=-1, keepdims=True)
    out_ref[...] = z - jnp.log(jnp.sum(jnp.exp(z), axis=-1, keepdims=True))


def _pad_edges(edge_index, n, e, bsz):
    ch = -(-e // (NW * bsz))
    ch += (-ch) % DEPTH
    e_pad = ch * NW * bsz
    src = edge_index[0]
    dst = edge_index[1]
    pad = e_pad - e
    if pad:
        src = jnp.concatenate([src, jnp.zeros((pad,), jnp.int32)])
        dst = jnp.concatenate([dst, jnp.full((pad,), n, jnp.int32)])
    edges = jnp.stack([src.reshape(NW, ch, bsz),
                       dst.reshape(NW, ch, bsz)], axis=2)
    return edges, ch


def kernel(edge_index, inputs, W_self0, W_neigh0, b0, gamma0, beta0,
           W_self1, W_neigh1, b1, gamma1, beta1, W_self2, W_neigh2, b2):
    n, d = inputs.shape
    e = edge_index.shape[1]
    d_out = W_self2.shape[1]

    n_pad = -(-(n + 1) // (NS * 8)) * (NS * 8)
    edges_a, ch_a = _pad_edges(edge_index, n, e, B_A)
    edges_b, ch_b = _pad_edges(edge_index, n, e, B_B)
    zero_d = jnp.zeros((n_pad, d), jnp.float32)
    zero_1 = jnp.zeros((n_pad,), jnp.float32)

    pass_a = _make_pass(n_pad, ch_a, d, B_A, 'a')
    pass_b = _make_pass(n_pad, ch_b, d, B_B, 'b')

    def layer(x, sums, degs, ws, wn, b, g, be, make_inv):
        outs = [jax.ShapeDtypeStruct((n, d), jnp.float32)]
        if make_inv:
            outs.append(jax.ShapeDtypeStruct((n_pad,), jnp.float32))
        return pl.pallas_call(
            functools.partial(_layer_body, make_inv, n, n_pad),
            out_shape=outs,
        )(x, sums, degs, ws, wn, b, g, be)

    sums_a, degv = pass_a(inputs, edges_a, zero_d, zero_1)
    h1, invd = layer(inputs, sums_a, degv, W_self0, W_neigh0, b0, gamma0,
                     beta0, True)
    sums_b, cv = pass_b(h1, invd, edges_b, zero_d, zero_1)
    (feat,) = layer(h1, sums_b, degv, W_self1, W_neigh1, b1, gamma1,
                    beta1, False)
    out = pl.pallas_call(
        functools.partial(_final_body, n),
        out_shape=jax.ShapeDtypeStruct((1, d_out), jnp.float32),
    )(feat, cv, W_self2, W_neigh2, b2)
    return out, inputs, feat
